# baseline (speedup 1.0000x reference)
import functools

import jax
import jax.numpy as jnp
from jax import lax
from jax.experimental import pallas as pl
from jax.experimental.pallas import tpu as pltpu
from jax.experimental.pallas import tpu_sc as plsc

_info = plsc.get_sparse_core_info()
_NC, _NS = _info.num_cores, _info.num_subcores
_NW = _NC * _NS

_COLS = 8192


def _transpose_body(tt_ref, out_ref):
    x = tt_ref[...]
    for s in range(x.shape[1] // 512):
        parts = [
            x[:, 512 * s + 128 * a : 512 * s + 128 * (a + 1)] for a in range(4)
        ]
        out_ref[pl.ds(128 * s, 128), :] = jnp.concatenate(parts, axis=0).T


def _make_tc_transpose(n_rows, dim):
    grid = (n_rows + _COLS - 1) // _COLS
    return pl.pallas_call(
        _transpose_body,
        grid=(grid,),
        in_specs=[pl.BlockSpec((dim, _COLS), lambda c: (0, c))],
        out_specs=pl.BlockSpec((_COLS * dim // 128, 128), lambda c: (c, 0)),
        out_shape=jax.ShapeDtypeStruct((grid * _COLS * dim // 128, 128), jnp.float32),
    )


def _make_sc_gather(batch, n_rows_padded, dim):
    b_per_w = batch // _NW
    mesh = plsc.VectorSubcoreMesh(core_axis_name="c", subcore_axis_name="s")

    @functools.partial(
        pl.kernel,
        mesh=mesh,
        out_type=jax.ShapeDtypeStruct((dim, batch), jnp.float32),
        scratch_types=[
            pltpu.VMEM((b_per_w,), jnp.int32),
            pltpu.VMEM((b_per_w,), jnp.int32),
            pltpu.VMEM((b_per_w, dim), jnp.float32),
            pltpu.VMEM((dim * b_per_w,), jnp.float32),
            pltpu.SemaphoreType.DMA,
        ],
        compiler_params=pltpu.CompilerParams(
            use_tc_tiling_on_sc=False, needs_layout_passes=False
        ),
    )
    def gather_kernel(table_hbm, idx_hbm, out_hbm, idx_v, jdx_v, rows_v, t_v, sem):
        wid = lax.axis_index("s") * _NC + lax.axis_index("c")
        base = wid * b_per_w
        pltpu.sync_copy(idx_hbm.at[pl.ds(base, b_per_w)], idx_v)
        for k in range(b_per_w // 16):
            i = idx_v[pl.ds(k * 16, 16)]
            j = (i - (i & 511)) + ((i & 127) << 2) + ((i >> 7) & 3)
            jdx_v[pl.ds(k * 16, 16)] = j
        pltpu.async_copy(table_hbm.at[jdx_v], rows_v, sem).wait()
        iota = lax.iota(jnp.int32, 16)

        def _txp(k, _):
            rows16 = k * 16 + iota
            for d in range(dim):
                v = plsc.load_gather(
                    rows_v, [rows16, jnp.full((16,), d, jnp.int32)]
                )
                t_v[pl.ds(d * b_per_w + k * 16, 16)] = v
            return ()

        lax.fori_loop(0, b_per_w // 16, _txp, (), unroll=2)
        for d in range(dim):
            pltpu.sync_copy(
                t_v.at[pl.ds(d * b_per_w, b_per_w)],
                out_hbm.at[d, pl.ds(base, b_per_w)],
            )

    return gather_kernel


@jax.jit
def kernel(ethnicity_idx, embedding_table):
    batch = ethnicity_idx.shape[0]
    n_rows, dim = embedding_table.shape
    idx = ethnicity_idx.astype(jnp.int32)
    t2 = _make_tc_transpose(n_rows, dim)(embedding_table.T)
    n_rows_padded = t2.shape[0] * 128 // dim
    t_lin = t2.reshape(n_rows_padded, dim)
    gather = _make_sc_gather(batch, n_rows_padded, dim)
    return gather(t_lin, idx).T


# speedup vs baseline: 1.1009x; 1.1009x over previous
import functools

import jax
import jax.numpy as jnp
from jax import lax
from jax.experimental import pallas as pl
from jax.experimental.pallas import tpu as pltpu
from jax.experimental.pallas import tpu_sc as plsc

_info = plsc.get_sparse_core_info()
_NC, _NS = _info.num_cores, _info.num_subcores
_NW = _NC * _NS

_COLS = 8192


def _transpose_body(tt_ref, out_ref):
    x = tt_ref[...]
    for s in range(x.shape[1] // 512):
        parts = [
            x[:, 512 * s + 128 * a : 512 * s + 128 * (a + 1)] for a in range(4)
        ]
        out_ref[pl.ds(128 * s, 128), :] = jnp.concatenate(parts, axis=0).T


def _make_tc_transpose(n_rows, dim):
    grid = (n_rows + _COLS - 1) // _COLS
    return pl.pallas_call(
        _transpose_body,
        grid=(grid,),
        in_specs=[pl.BlockSpec((dim, _COLS), lambda c: (0, c))],
        out_specs=pl.BlockSpec((_COLS * dim // 128, 128), lambda c: (c, 0)),
        out_shape=jax.ShapeDtypeStruct((grid * _COLS * dim // 128, 128), jnp.float32),
    )


def _make_sc_gather(batch, n_rows_padded, dim):
    b_per_w = batch // _NW
    mesh = plsc.VectorSubcoreMesh(core_axis_name="c", subcore_axis_name="s")

    @functools.partial(
        pl.kernel,
        mesh=mesh,
        out_type=jax.ShapeDtypeStruct((batch, dim), jnp.float32),
        scratch_types=[
            pltpu.VMEM((b_per_w,), jnp.int32),
            pltpu.VMEM((b_per_w,), jnp.int32),
            pltpu.VMEM((b_per_w, dim), jnp.float32),
            pltpu.SemaphoreType.DMA,
        ],
        compiler_params=pltpu.CompilerParams(use_tc_tiling_on_sc=False),
    )
    def gather_kernel(table_hbm, idx_hbm, out_hbm, idx_v, jdx_v, rows_v, sem):
        wid = lax.axis_index("s") * _NC + lax.axis_index("c")
        base = wid * b_per_w
        pltpu.sync_copy(idx_hbm.at[pl.ds(base, b_per_w)], idx_v)
        for k in range(b_per_w // 16):
            i = idx_v[pl.ds(k * 16, 16)]
            j = (i - (i & 511)) + ((i & 127) << 2) + ((i >> 7) & 3)
            jdx_v[pl.ds(k * 16, 16)] = j
        pltpu.async_copy(table_hbm.at[jdx_v], rows_v, sem).wait()
        pltpu.sync_copy(rows_v, out_hbm.at[pl.ds(base, b_per_w)])

    return gather_kernel


@jax.jit
def kernel(ethnicity_idx, embedding_table):
    batch = ethnicity_idx.shape[0]
    n_rows, dim = embedding_table.shape
    idx = ethnicity_idx.astype(jnp.int32)
    t2 = _make_tc_transpose(n_rows, dim)(embedding_table.T)
    n_rows_padded = t2.shape[0] * 128 // dim
    t_lin = t2.reshape(n_rows_padded, dim)
    gather = _make_sc_gather(batch, n_rows_padded, dim)
    return gather(t_lin, idx)


# speedup vs baseline: 1.1568x; 1.0508x over previous
import functools

import jax
import jax.numpy as jnp
from jax import lax
from jax.experimental import pallas as pl
from jax.experimental.pallas import tpu as pltpu
from jax.experimental.pallas import tpu_sc as plsc

_info = plsc.get_sparse_core_info()
_NC, _NS = _info.num_cores, _info.num_subcores
_NW = _NC * _NS

_COLS = 10240


def _transpose_body(tt_ref, out_ref):
    x = tt_ref[...]
    for s in range(x.shape[1] // 512):
        parts = [
            x[:, 512 * s + 128 * a : 512 * s + 128 * (a + 1)] for a in range(4)
        ]
        out_ref[pl.ds(128 * s, 128), :] = jnp.concatenate(parts, axis=0).T


def _make_tc_transpose(n_rows, dim):
    grid = (n_rows + _COLS - 1) // _COLS
    return pl.pallas_call(
        _transpose_body,
        grid=(grid,),
        in_specs=[pl.BlockSpec((dim, _COLS), lambda c: (0, c))],
        out_specs=pl.BlockSpec((_COLS * dim // 128, 128), lambda c: (c, 0)),
        out_shape=jax.ShapeDtypeStruct((grid * _COLS * dim // 128, 128), jnp.float32),
    )


def _make_sc_gather(batch, n_rows_padded, dim):
    b_per_w = batch // _NW
    mesh = plsc.VectorSubcoreMesh(core_axis_name="c", subcore_axis_name="s")

    @functools.partial(
        pl.kernel,
        mesh=mesh,
        out_type=jax.ShapeDtypeStruct((batch, dim), jnp.float32),
        scratch_types=[
            pltpu.VMEM((b_per_w,), jnp.int32),
            pltpu.VMEM((b_per_w,), jnp.int32),
            pltpu.VMEM((b_per_w, dim), jnp.float32),
            pltpu.SemaphoreType.DMA,
        ],
        compiler_params=pltpu.CompilerParams(use_tc_tiling_on_sc=False),
    )
    def gather_kernel(table_hbm, idx_hbm, out_hbm, idx_v, jdx_v, rows_v, sem):
        wid = lax.axis_index("s") * _NC + lax.axis_index("c")
        base = wid * b_per_w
        pltpu.sync_copy(idx_hbm.at[pl.ds(base, b_per_w)], idx_v)
        for k in range(b_per_w // 16):
            i = idx_v[pl.ds(k * 16, 16)]
            j = (i - (i & 511)) + ((i & 127) << 2) + ((i >> 7) & 3)
            jdx_v[pl.ds(k * 16, 16)] = j
        pltpu.async_copy(table_hbm.at[jdx_v], rows_v, sem).wait()
        pltpu.sync_copy(rows_v, out_hbm.at[pl.ds(base, b_per_w)])

    return gather_kernel


@jax.jit
def kernel(ethnicity_idx, embedding_table):
    batch = ethnicity_idx.shape[0]
    n_rows, dim = embedding_table.shape
    idx = ethnicity_idx.astype(jnp.int32)
    t2 = _make_tc_transpose(n_rows, dim)(embedding_table.T)
    n_rows_padded = t2.shape[0] * 128 // dim
    t_lin = t2.reshape(n_rows_padded, dim)
    gather = _make_sc_gather(batch, n_rows_padded, dim)
    return gather(t_lin, idx)
